# baseline (speedup 1.0000x reference)
import functools

import jax
import jax.numpy as jnp
from jax import lax
from jax.experimental import pallas as pl
from jax.experimental.pallas import tpu as pltpu
from jax.experimental.pallas import tpu_sc as plsc

IMG = 512
PS = 16
C = 384
L = (IMG // PS) ** 2
BN = 32
D = PS * PS * 3
S_NUM = 256
SL_NUM = 768
L_NUM = SL_NUM - S_NUM
T_NUM = L - SL_NUM

NC = 2
NS = 16
NW = NC * NS
CH = 64


def _rowsum(m):
    ones = jnp.ones((m.shape[1], 8), jnp.float32)
    s = lax.dot_general(m, ones, (((1,), (0,)), ((), ())),
                        precision=lax.Precision.HIGHEST,
                        preferred_element_type=jnp.float32)
    return lax.slice(s, (0, 0), (m.shape[0], 1))


def _colsum(m):
    ones = jnp.ones((8, m.shape[0]), jnp.float32)
    s = lax.dot_general(ones, m, (((1,), (0,)), ((), ())),
                        precision=lax.Precision.HIGHEST,
                        preferred_element_type=jnp.float32)
    return lax.slice(s, (0, 0), (1, m.shape[1]))


def _index_kernel(noise_row_ref, noise_col_ref,
                  gs_ref, gl_ref, gt_ref, ps_ref, pl_ref, pt_ref, flag_ref):
    b = pl.program_id(0)
    v_row = noise_row_ref[0]
    v_col = noise_col_ref[0]

    vr = jnp.broadcast_to(v_row, (L, L))
    vc = jnp.broadcast_to(v_col, (L, L))
    ii = lax.broadcasted_iota(jnp.int32, (L, L), 0)
    jj = lax.broadcasted_iota(jnp.int32, (L, L), 1)
    m = ((vr < vc) | ((vr == vc) & (jj < ii))).astype(jnp.float32)
    rank_col = _rowsum(m)
    rank_row = (L - 1) - _colsum(m)

    def sorted_subset(lo, n):
        rc = lax.slice(rank_col, (lo, 0), (lo + n, 1))
        rr = lax.slice(rank_row, (0, lo), (1, lo + n))
        less = (jnp.broadcast_to(rr, (n, n)) <
                jnp.broadcast_to(rc, (n, n))).astype(jnp.float32)
        slot = (_rowsum(less) + 0.5).astype(jnp.int32)
        kk = lax.broadcasted_iota(jnp.int32, (n, n), 1)
        onehot = (jnp.broadcast_to(slot, (n, n)) == kk).astype(jnp.float32)
        vals = onehot * jnp.broadcast_to(rc, (n, n))
        return _colsum(vals) + 0.5

    idx_s = sorted_subset(0, S_NUM)
    idx_l = sorted_subset(S_NUM, L_NUM)
    idx_t = sorted_subset(SL_NUM, T_NUM)

    base = (b * L).astype(jnp.float32)
    gs_ref[0] = (idx_s + base).astype(jnp.int32)
    gl_ref[0] = (idx_l + base).astype(jnp.int32)
    gt_ref[0] = (idx_t + base).astype(jnp.int32)
    ps_ref[0] = (idx_s + 1.0).astype(jnp.int32)
    pl_ref[0] = (idx_l + 1.0).astype(jnp.int32)
    pt_ref[0] = (idx_t + 1.0).astype(jnp.int32)

    rc_s = lax.slice(rank_col, (0, 0), (S_NUM, 1))
    rr_sl = lax.slice(rank_row, (0, 0), (1, SL_NUM))
    less = (jnp.broadcast_to(rr_sl, (S_NUM, SL_NUM)) <
            jnp.broadcast_to(rc_s, (S_NUM, SL_NUM))).astype(jnp.float32)
    slot = (_rowsum(less) + 0.5).astype(jnp.int32)
    kk = lax.broadcasted_iota(jnp.int32, (S_NUM, SL_NUM), 1)
    onehot = (jnp.broadcast_to(slot, (S_NUM, SL_NUM)) == kk).astype(jnp.float32)
    flags = _colsum(onehot)
    flag_ref[0] = (flags > 0.5).astype(jnp.int32)


def _compute_indices(noise):
    nrow = noise.reshape(BN, 1, L)
    ncol = noise.reshape(BN, L, 1)
    outs = pl.pallas_call(
        _index_kernel,
        grid=(BN,),
        in_specs=[
            pl.BlockSpec((1, 1, L), lambda b: (b, 0, 0)),
            pl.BlockSpec((1, L, 1), lambda b: (b, 0, 0)),
        ],
        out_specs=[
            pl.BlockSpec((1, 1, S_NUM), lambda b: (b, 0, 0)),
            pl.BlockSpec((1, 1, L_NUM), lambda b: (b, 0, 0)),
            pl.BlockSpec((1, 1, T_NUM), lambda b: (b, 0, 0)),
            pl.BlockSpec((1, 1, S_NUM), lambda b: (b, 0, 0)),
            pl.BlockSpec((1, 1, L_NUM), lambda b: (b, 0, 0)),
            pl.BlockSpec((1, 1, T_NUM), lambda b: (b, 0, 0)),
            pl.BlockSpec((1, 1, SL_NUM), lambda b: (b, 0, 0)),
        ],
        out_shape=[
            jax.ShapeDtypeStruct((BN, 1, S_NUM), jnp.int32),
            jax.ShapeDtypeStruct((BN, 1, L_NUM), jnp.int32),
            jax.ShapeDtypeStruct((BN, 1, T_NUM), jnp.int32),
            jax.ShapeDtypeStruct((BN, 1, S_NUM), jnp.int32),
            jax.ShapeDtypeStruct((BN, 1, L_NUM), jnp.int32),
            jax.ShapeDtypeStruct((BN, 1, T_NUM), jnp.int32),
            jax.ShapeDtypeStruct((BN, 1, SL_NUM), jnp.int32),
        ],
    )(nrow, ncol)
    return outs


def _sc_gather_kernel(ps_tab, pt_tab, tl_tab, poss_tab, post_tab,
                      gl_idx, gs_idx, gt_idx, pl_idx, ps_idx, pt_idx,
                      tgt_out, gs_out, gt_out, lp_out, pgs_out, pgt_out,
                      idx_v0, idx_v1, buf_d0, buf_d1, buf_c0,
                      sem0, sem1):
    wid = lax.axis_index("s") * NC + lax.axis_index("c")
    ivecs = (idx_v0, idx_v1)

    def job(table, idx_hbm, out_hbm, bufs, sems, rows_per_w):
        nchunks = rows_per_w // CH
        base = wid * rows_per_w

        def fire(i, slot):
            start = base + i * CH
            pltpu.sync_copy(idx_hbm.at[pl.ds(start, CH)], ivecs[slot])
            return pltpu.async_copy(table.at[ivecs[slot]], bufs[slot],
                                    sems[slot])

        def drain(i, slot, handle):
            start = base + i * CH
            handle.wait()
            pltpu.sync_copy(bufs[slot], out_hbm.at[pl.ds(start, CH)])

        h_prev = fire(0, 0)
        for k in range(1, nchunks):
            h_cur = fire(k, k % 2)
            drain(k - 1, (k - 1) % 2, h_prev)
            h_prev = h_cur
        drain(nchunks - 1, (nchunks - 1) % 2, h_prev)

    def job_seq(table, idx_hbm, out_hbm, buf, sem, rows_per_w):
        nchunks = rows_per_w // CH
        base = wid * rows_per_w

        def body(i, carry):
            start = base + i * CH
            pltpu.sync_copy(idx_hbm.at[pl.ds(start, CH)], idx_v0)
            pltpu.async_copy(table.at[idx_v0], buf, sem).wait()
            pltpu.sync_copy(buf, out_hbm.at[pl.ds(start, CH)])
            return carry

        lax.fori_loop(0, nchunks, body, 0)

    job(ps_tab, gl_idx, tgt_out, (buf_d0, buf_d1), (sem0, sem1),
        (BN * L_NUM) // NW)
    job(ps_tab, gs_idx, gs_out, (buf_d0, buf_d1), (sem0, sem1),
        (BN * S_NUM) // NW)
    job(pt_tab, gt_idx, gt_out, (buf_d0, buf_d1), (sem0, sem1),
        (BN * T_NUM) // NW)
    job_seq(tl_tab, pl_idx, lp_out, buf_c0, sem0, (BN * L_NUM) // NW)
    job_seq(poss_tab, ps_idx, pgs_out, buf_c0, sem0, (BN * S_NUM) // NW)
    job_seq(post_tab, pt_idx, pgt_out, buf_c0, sem0, (BN * T_NUM) // NW)


def _sc_gather(ps_tab, pt_tab, tl_tab, poss_tab, post_tab,
               gl_idx, gs_idx, gt_idx, pl_idx, ps_idx, pt_idx):
    mesh = plsc.VectorSubcoreMesh(core_axis_name="c", subcore_axis_name="s")
    f32 = jnp.float32
    kern = functools.partial(
        pl.kernel,
        mesh=mesh,
        out_type=[
            jax.ShapeDtypeStruct((BN * L_NUM, D), f32),
            jax.ShapeDtypeStruct((BN * S_NUM, D), f32),
            jax.ShapeDtypeStruct((BN * T_NUM, D), f32),
            jax.ShapeDtypeStruct((BN * L_NUM, C), f32),
            jax.ShapeDtypeStruct((BN * S_NUM, C), f32),
            jax.ShapeDtypeStruct((BN * T_NUM, C), f32),
        ],
        scratch_types=[
            pltpu.VMEM((CH,), jnp.int32),
            pltpu.VMEM((CH,), jnp.int32),
            pltpu.VMEM((CH, D), f32),
            pltpu.VMEM((CH, D), f32),
            pltpu.VMEM((CH, C), f32),
            pltpu.SemaphoreType.DMA,
            pltpu.SemaphoreType.DMA,
        ],
    )(_sc_gather_kernel)
    return kern(ps_tab, pt_tab, tl_tab, poss_tab, post_tab,
                gl_idx, gs_idx, gt_idx, pl_idx, ps_idx, pt_idx)


def _embed_kernel(gs_ref, gt_ref, pgs_ref, pgt_ref, w_ref, b_ref,
                  s0_ref, t0_ref, souts_ref, touts_ref):
    w = w_ref[...]
    bias = b_ref[...]
    es = lax.dot_general(gs_ref[0], w, (((1,), (0,)), ((), ())),
                         precision=lax.Precision.HIGHEST,
                         preferred_element_type=jnp.float32)
    souts_ref[0] = jnp.concatenate([s0_ref[...], es + bias + pgs_ref[0]],
                                   axis=0)
    et = lax.dot_general(gt_ref[0], w, (((1,), (0,)), ((), ())),
                         precision=lax.Precision.HIGHEST,
                         preferred_element_type=jnp.float32)
    touts_ref[0] = jnp.concatenate([t0_ref[...], et + bias + pgt_ref[0]],
                                   axis=0)


def _embed(gs, gt, pgs, pgt, w, bias, s0, t0):
    return pl.pallas_call(
        _embed_kernel,
        grid=(BN,),
        in_specs=[
            pl.BlockSpec((1, S_NUM, D), lambda b: (b, 0, 0)),
            pl.BlockSpec((1, T_NUM, D), lambda b: (b, 0, 0)),
            pl.BlockSpec((1, S_NUM, C), lambda b: (b, 0, 0)),
            pl.BlockSpec((1, T_NUM, C), lambda b: (b, 0, 0)),
            pl.BlockSpec((D, C), lambda b: (0, 0)),
            pl.BlockSpec((1, C), lambda b: (0, 0)),
            pl.BlockSpec((1, C), lambda b: (0, 0)),
            pl.BlockSpec((1, C), lambda b: (0, 0)),
        ],
        out_specs=[
            pl.BlockSpec((1, S_NUM + 1, C), lambda b: (b, 0, 0)),
            pl.BlockSpec((1, T_NUM + 1, C), lambda b: (b, 0, 0)),
        ],
        out_shape=[
            jax.ShapeDtypeStruct((BN, S_NUM + 1, C), jnp.float32),
            jax.ShapeDtypeStruct((BN, T_NUM + 1, C), jnp.float32),
        ],
    )(gs, gt, pgs, pgt, w, bias, s0, t0)


def _patchify(imgs):
    n, c, hh, ww = imgs.shape
    h = hh // PS
    w = ww // PS
    x = imgs.reshape(n, c, h, PS, w, PS)
    x = jnp.einsum('nchpwq->nhwpqc', x)
    return x.reshape(n, h * w, PS * PS * c)


def kernel(s_crops, t_crops, noise, W_patch, b_patch, cls_s, cls_t,
           pos_s, pos_t, s_mask_token):
    gs3, gl3, gt3, ps3, pl3, pt3, flag3 = _compute_indices(noise)
    gs_idx = gs3.reshape(BN * S_NUM)
    gl_idx = gl3.reshape(BN * L_NUM)
    gt_idx = gt3.reshape(BN * T_NUM)
    ps_idx = ps3.reshape(BN * S_NUM)
    pl_idx = pl3.reshape(BN * L_NUM)
    pt_idx = pt3.reshape(BN * T_NUM)

    ps_tab = _patchify(s_crops).reshape(BN * L, D)
    pt_tab = _patchify(t_crops).reshape(BN * L, D)
    poss_tab = pos_s.reshape(L + 1, C)
    post_tab = pos_t.reshape(L + 1, C)
    tl_tab = (s_mask_token + pos_s).reshape(L + 1, C)

    tgt, gs, gt, lp, pgs, pgt = _sc_gather(
        ps_tab, pt_tab, tl_tab, poss_tab, post_tab,
        gl_idx, gs_idx, gt_idx, pl_idx, ps_idx, pt_idx)

    s0 = (cls_s + pos_s[:, :1]).reshape(1, C)
    t0 = (cls_t + pos_t[:, :1]).reshape(1, C)
    s_out, t_out = _embed(gs.reshape(BN, S_NUM, D), gt.reshape(BN, T_NUM, D),
                          pgs.reshape(BN, S_NUM, C), pgt.reshape(BN, T_NUM, C),
                          W_patch, b_patch.reshape(1, C), s0, t0)

    l_patches = lp.reshape(BN, L_NUM, C)
    s_global_target = tgt.reshape(BN, L_NUM, D)
    ones_col = jnp.ones((BN, 1), dtype=bool)
    cs_mask = jnp.concatenate(
        [ones_col, (flag3.reshape(BN, SL_NUM) > 0)], axis=1)
    return (s_out, t_out, l_patches, s_global_target, cs_mask)


# speedup vs baseline: 1.1931x; 1.1931x over previous
import functools

import jax
import jax.numpy as jnp
from jax import lax
from jax.experimental import pallas as pl
from jax.experimental.pallas import tpu as pltpu
from jax.experimental.pallas import tpu_sc as plsc

IMG = 512
PS = 16
C = 384
L = (IMG // PS) ** 2
BN = 32
D = PS * PS * 3
S_NUM = 256
SL_NUM = 768
L_NUM = SL_NUM - S_NUM
T_NUM = L - SL_NUM

NC = 2
NS = 16
NW = NC * NS
CH = 64


def _index_kernel(noise_row_ref, noise_col_ref,
                  gs_ref, gl_ref, gt_ref, ps_ref, pl_ref, pt_ref, flag_ref):
    b = pl.program_id(0)
    v_row = noise_row_ref[0]
    v_col = noise_col_ref[0]

    vr = jnp.broadcast_to(v_row, (L, L))
    vc = jnp.broadcast_to(v_col, (L, L))
    ii = lax.broadcasted_iota(jnp.int32, (L, L), 0)
    jj = lax.broadcasted_iota(jnp.int32, (L, L), 1)
    m = ((vr < vc) | ((vr == vc) & (jj < ii))).astype(jnp.float32)
    rank_col = jnp.sum(m, axis=1, keepdims=True)
    rank_row = (L - 1) - jnp.sum(m, axis=0, keepdims=True)

    def sorted_subset(lo, n):
        rc = lax.slice(rank_col, (lo, 0), (lo + n, 1))
        rr = lax.slice(rank_row, (0, lo), (1, lo + n))
        less = (jnp.broadcast_to(rr, (n, n)) <
                jnp.broadcast_to(rc, (n, n))).astype(jnp.float32)
        slot = jnp.sum(less, axis=1, keepdims=True).astype(jnp.int32)
        kk = lax.broadcasted_iota(jnp.int32, (n, n), 1)
        onehot = (jnp.broadcast_to(slot, (n, n)) == kk).astype(jnp.float32)
        vals = onehot * jnp.broadcast_to(rc, (n, n))
        return jnp.sum(vals, axis=0, keepdims=True)

    idx_s = sorted_subset(0, S_NUM)
    idx_l = sorted_subset(S_NUM, L_NUM)
    idx_t = sorted_subset(SL_NUM, T_NUM)

    base = (b * L).astype(jnp.float32)
    gs_ref[0] = (idx_s + base).astype(jnp.int32)
    gl_ref[0] = (idx_l + base).astype(jnp.int32)
    gt_ref[0] = (idx_t + base).astype(jnp.int32)
    ps_ref[0] = (idx_s + 1.0).astype(jnp.int32)
    pl_ref[0] = (idx_l + 1.0).astype(jnp.int32)
    pt_ref[0] = (idx_t + 1.0).astype(jnp.int32)

    rc_s = lax.slice(rank_col, (0, 0), (S_NUM, 1))
    rr_sl = lax.slice(rank_row, (0, 0), (1, SL_NUM))
    less = (jnp.broadcast_to(rr_sl, (S_NUM, SL_NUM)) <
            jnp.broadcast_to(rc_s, (S_NUM, SL_NUM))).astype(jnp.float32)
    slot = jnp.sum(less, axis=1, keepdims=True).astype(jnp.int32)
    kk = lax.broadcasted_iota(jnp.int32, (S_NUM, SL_NUM), 1)
    onehot = (jnp.broadcast_to(slot, (S_NUM, SL_NUM)) == kk).astype(jnp.float32)
    flags = jnp.sum(onehot, axis=0, keepdims=True)
    flag_ref[0] = (flags > 0.5).astype(jnp.int32)


def _compute_indices(noise):
    nrow = noise.reshape(BN, 1, L)
    ncol = noise.reshape(BN, L, 1)
    outs = pl.pallas_call(
        _index_kernel,
        grid=(BN,),
        in_specs=[
            pl.BlockSpec((1, 1, L), lambda b: (b, 0, 0)),
            pl.BlockSpec((1, L, 1), lambda b: (b, 0, 0)),
        ],
        out_specs=[
            pl.BlockSpec((1, 1, S_NUM), lambda b: (b, 0, 0)),
            pl.BlockSpec((1, 1, L_NUM), lambda b: (b, 0, 0)),
            pl.BlockSpec((1, 1, T_NUM), lambda b: (b, 0, 0)),
            pl.BlockSpec((1, 1, S_NUM), lambda b: (b, 0, 0)),
            pl.BlockSpec((1, 1, L_NUM), lambda b: (b, 0, 0)),
            pl.BlockSpec((1, 1, T_NUM), lambda b: (b, 0, 0)),
            pl.BlockSpec((1, 1, SL_NUM), lambda b: (b, 0, 0)),
        ],
        out_shape=[
            jax.ShapeDtypeStruct((BN, 1, S_NUM), jnp.int32),
            jax.ShapeDtypeStruct((BN, 1, L_NUM), jnp.int32),
            jax.ShapeDtypeStruct((BN, 1, T_NUM), jnp.int32),
            jax.ShapeDtypeStruct((BN, 1, S_NUM), jnp.int32),
            jax.ShapeDtypeStruct((BN, 1, L_NUM), jnp.int32),
            jax.ShapeDtypeStruct((BN, 1, T_NUM), jnp.int32),
            jax.ShapeDtypeStruct((BN, 1, SL_NUM), jnp.int32),
        ],
    )(nrow, ncol)
    return outs


def _sc_gather_kernel(ps_tab, pt_tab, tl_tab, poss_tab, post_tab,
                      gl_idx, gs_idx, gt_idx, pl_idx, ps_idx, pt_idx,
                      tgt_out, gs_out, gt_out, lp_out, pgs_out, pgt_out,
                      idx_v0, idx_v1, buf_d0, buf_d1, buf_c0,
                      sem0, sem1):
    wid = lax.axis_index("s") * NC + lax.axis_index("c")
    ivecs = (idx_v0, idx_v1)

    def job(table, idx_hbm, out_hbm, bufs, sems, rows_per_w):
        nchunks = rows_per_w // CH
        base = wid * rows_per_w

        def fire(i, slot):
            start = base + i * CH
            pltpu.sync_copy(idx_hbm.at[pl.ds(start, CH)], ivecs[slot])
            return pltpu.async_copy(table.at[ivecs[slot]], bufs[slot],
                                    sems[slot])

        def drain(i, slot, handle):
            start = base + i * CH
            handle.wait()
            pltpu.sync_copy(bufs[slot], out_hbm.at[pl.ds(start, CH)])

        h_prev = fire(0, 0)
        for k in range(1, nchunks):
            h_cur = fire(k, k % 2)
            drain(k - 1, (k - 1) % 2, h_prev)
            h_prev = h_cur
        drain(nchunks - 1, (nchunks - 1) % 2, h_prev)

    def job_seq(table, idx_hbm, out_hbm, buf, sem, rows_per_w):
        nchunks = rows_per_w // CH
        base = wid * rows_per_w

        def body(i, carry):
            start = base + i * CH
            pltpu.sync_copy(idx_hbm.at[pl.ds(start, CH)], idx_v0)
            pltpu.async_copy(table.at[idx_v0], buf, sem).wait()
            pltpu.sync_copy(buf, out_hbm.at[pl.ds(start, CH)])
            return carry

        lax.fori_loop(0, nchunks, body, 0)

    job(ps_tab, gl_idx, tgt_out, (buf_d0, buf_d1), (sem0, sem1),
        (BN * L_NUM) // NW)
    job(ps_tab, gs_idx, gs_out, (buf_d0, buf_d1), (sem0, sem1),
        (BN * S_NUM) // NW)
    job(pt_tab, gt_idx, gt_out, (buf_d0, buf_d1), (sem0, sem1),
        (BN * T_NUM) // NW)
    job_seq(tl_tab, pl_idx, lp_out, buf_c0, sem0, (BN * L_NUM) // NW)
    job_seq(poss_tab, ps_idx, pgs_out, buf_c0, sem0, (BN * S_NUM) // NW)
    job_seq(post_tab, pt_idx, pgt_out, buf_c0, sem0, (BN * T_NUM) // NW)


def _sc_gather(ps_tab, pt_tab, tl_tab, poss_tab, post_tab,
               gl_idx, gs_idx, gt_idx, pl_idx, ps_idx, pt_idx):
    mesh = plsc.VectorSubcoreMesh(core_axis_name="c", subcore_axis_name="s")
    f32 = jnp.float32
    kern = functools.partial(
        pl.kernel,
        mesh=mesh,
        out_type=[
            jax.ShapeDtypeStruct((BN * L_NUM, D), f32),
            jax.ShapeDtypeStruct((BN * S_NUM, D), f32),
            jax.ShapeDtypeStruct((BN * T_NUM, D), f32),
            jax.ShapeDtypeStruct((BN * L_NUM, C), f32),
            jax.ShapeDtypeStruct((BN * S_NUM, C), f32),
            jax.ShapeDtypeStruct((BN * T_NUM, C), f32),
        ],
        scratch_types=[
            pltpu.VMEM((CH,), jnp.int32),
            pltpu.VMEM((CH,), jnp.int32),
            pltpu.VMEM((CH, D), f32),
            pltpu.VMEM((CH, D), f32),
            pltpu.VMEM((CH, C), f32),
            pltpu.SemaphoreType.DMA,
            pltpu.SemaphoreType.DMA,
        ],
    )(_sc_gather_kernel)
    return kern(ps_tab, pt_tab, tl_tab, poss_tab, post_tab,
                gl_idx, gs_idx, gt_idx, pl_idx, ps_idx, pt_idx)


def _embed_kernel(gs_ref, gt_ref, pgs_ref, pgt_ref, w_ref, b_ref,
                  s0_ref, t0_ref, souts_ref, touts_ref):
    w = w_ref[...]
    bias = b_ref[...]
    es = lax.dot_general(gs_ref[0], w, (((1,), (0,)), ((), ())),
                         precision=lax.Precision.HIGHEST,
                         preferred_element_type=jnp.float32)
    souts_ref[0] = jnp.concatenate([s0_ref[...], es + bias + pgs_ref[0]],
                                   axis=0)
    et = lax.dot_general(gt_ref[0], w, (((1,), (0,)), ((), ())),
                         precision=lax.Precision.HIGHEST,
                         preferred_element_type=jnp.float32)
    touts_ref[0] = jnp.concatenate([t0_ref[...], et + bias + pgt_ref[0]],
                                   axis=0)


def _embed(gs, gt, pgs, pgt, w, bias, s0, t0):
    return pl.pallas_call(
        _embed_kernel,
        grid=(BN,),
        in_specs=[
            pl.BlockSpec((1, S_NUM, D), lambda b: (b, 0, 0)),
            pl.BlockSpec((1, T_NUM, D), lambda b: (b, 0, 0)),
            pl.BlockSpec((1, S_NUM, C), lambda b: (b, 0, 0)),
            pl.BlockSpec((1, T_NUM, C), lambda b: (b, 0, 0)),
            pl.BlockSpec((D, C), lambda b: (0, 0)),
            pl.BlockSpec((1, C), lambda b: (0, 0)),
            pl.BlockSpec((1, C), lambda b: (0, 0)),
            pl.BlockSpec((1, C), lambda b: (0, 0)),
        ],
        out_specs=[
            pl.BlockSpec((1, S_NUM + 1, C), lambda b: (b, 0, 0)),
            pl.BlockSpec((1, T_NUM + 1, C), lambda b: (b, 0, 0)),
        ],
        out_shape=[
            jax.ShapeDtypeStruct((BN, S_NUM + 1, C), jnp.float32),
            jax.ShapeDtypeStruct((BN, T_NUM + 1, C), jnp.float32),
        ],
    )(gs, gt, pgs, pgt, w, bias, s0, t0)


def _patchify(imgs):
    n, c, hh, ww = imgs.shape
    h = hh // PS
    w = ww // PS
    x = imgs.reshape(n, c, h, PS, w, PS)
    x = jnp.einsum('nchpwq->nhwpqc', x)
    return x.reshape(n, h * w, PS * PS * c)


def kernel(s_crops, t_crops, noise, W_patch, b_patch, cls_s, cls_t,
           pos_s, pos_t, s_mask_token):
    gs3, gl3, gt3, ps3, pl3, pt3, flag3 = _compute_indices(noise)
    gs_idx = gs3.reshape(BN * S_NUM)
    gl_idx = gl3.reshape(BN * L_NUM)
    gt_idx = gt3.reshape(BN * T_NUM)
    ps_idx = ps3.reshape(BN * S_NUM)
    pl_idx = pl3.reshape(BN * L_NUM)
    pt_idx = pt3.reshape(BN * T_NUM)

    ps_tab = _patchify(s_crops).reshape(BN * L, D)
    pt_tab = _patchify(t_crops).reshape(BN * L, D)
    poss_tab = pos_s.reshape(L + 1, C)
    post_tab = pos_t.reshape(L + 1, C)
    tl_tab = (s_mask_token + pos_s).reshape(L + 1, C)

    tgt, gs, gt, lp, pgs, pgt = _sc_gather(
        ps_tab, pt_tab, tl_tab, poss_tab, post_tab,
        gl_idx, gs_idx, gt_idx, pl_idx, ps_idx, pt_idx)

    s0 = (cls_s + pos_s[:, :1]).reshape(1, C)
    t0 = (cls_t + pos_t[:, :1]).reshape(1, C)
    s_out, t_out = _embed(gs.reshape(BN, S_NUM, D), gt.reshape(BN, T_NUM, D),
                          pgs.reshape(BN, S_NUM, C), pgt.reshape(BN, T_NUM, C),
                          W_patch, b_patch.reshape(1, C), s0, t0)

    l_patches = lp.reshape(BN, L_NUM, C)
    s_global_target = tgt.reshape(BN, L_NUM, D)
    ones_col = jnp.ones((BN, 1), dtype=bool)
    cs_mask = jnp.concatenate(
        [ones_col, (flag3.reshape(BN, SL_NUM) > 0)], axis=1)
    return (s_out, t_out, l_patches, s_global_target, cs_mask)
